# baseline (speedup 1.0000x reference)
import jax
import jax.numpy as jnp
from jax import lax
from jax.experimental import pallas as pl
from jax.experimental.pallas import tpu as pltpu
from jax.experimental.pallas import tpu_sc as plsc
import functools

NUM_ENT = 1000000
NUM_REL = 1000
D = 64
MARGIN = 12.0
BATCH = 4096
ERANGE = (MARGIN + 2.0) / D
PI = 3.141592653589793
PHASE_SCALE = PI / ERANGE

NC, NS, L = 2, 16, 16
NW = NC * NS
BPW = BATCH // NW

_SIN_C = (0.9994501730582466, -0.16583842947680993, 0.007998575320167352,
          -0.00014774043807849746)
_COS_C = (0.9999710932183866, -0.49983759608552286, 0.04152230455014086,
          -0.0013441068677407103, 1.906521608691092e-05)


def _sincos(ph):
    x2 = ph * ph
    s = _SIN_C[-1]
    for c in _SIN_C[-2::-1]:
        s = s * x2 + c
    s = s * ph
    c_ = _COS_C[-1]
    for c in _COS_C[-2::-1]:
        c_ = c_ * x2 + c
    return s, c_


_sc_mesh = plsc.VectorSubcoreMesh(core_axis_name="c", subcore_axis_name="s")


@functools.partial(
    pl.kernel,
    out_type=jax.ShapeDtypeStruct((BATCH,), jnp.float32),
    mesh=_sc_mesh,
    compiler_params=pltpu.CompilerParams(needs_layout_passes=False,
                                         use_tc_tiling_on_sc=False),
    scratch_types=[
        pltpu.VMEM((BPW,), jnp.int32),
        pltpu.VMEM((BPW,), jnp.int32),
        pltpu.VMEM((BPW,), jnp.int32),
        pltpu.VMEM((BPW, 2 * D), jnp.float32),
        pltpu.VMEM((BPW, 2 * D), jnp.float32),
        pltpu.VMEM((BPW, D), jnp.float32),
        pltpu.VMEM((BPW + L,), jnp.float32),
        pltpu.SemaphoreType.DMA,
        pltpu.SemaphoreType.DMA,
        pltpu.SemaphoreType.DMA,
    ],
)
def _sc_score(sub_hbm, rel_hbm, obj_hbm, ent_hbm, rel_emb_hbm, out_hbm,
              sub_v, obj_v, rel_v, h_v, t_v, tab_v, o_v,
              sem_a, sem_b, sem_i):
    wid = lax.axis_index("s") * NC + lax.axis_index("c")
    base = wid * BPW
    bs = pl.ds(base, BPW)
    idx_copies = [pltpu.async_copy(sub_hbm.at[bs], sub_v, sem_i),
                  pltpu.async_copy(obj_hbm.at[bs], obj_v, sem_i),
                  pltpu.async_copy(rel_hbm.at[bs], rel_v, sem_i)]
    for c in idx_copies:
        c.wait()
    HALF = BPW // 2
    half = (pl.ds(0, HALF), pl.ds(HALF, HALF))
    copies = []
    for p, sem in ((0, sem_a), (1, sem_b)):
        copies += [
            pltpu.async_copy(ent_hbm.at[sub_v.at[half[p]]],
                             h_v.at[half[p]], sem),
            pltpu.async_copy(ent_hbm.at[obj_v.at[half[p]]],
                             t_v.at[half[p]], sem),
            pltpu.async_copy(rel_emb_hbm.at[rel_v.at[half[p]]],
                             tab_v.at[half[p]], sem),
        ]

    lane = lax.iota(jnp.int32, L)
    last = lane == (L - 1)

    def one_elem(b):
        acc = jnp.zeros((L,), jnp.float32)
        for k in range(D // L):
            sl = pl.ds(k * L, L)
            sl2 = pl.ds(D + k * L, L)
            re_h = h_v[b, sl]
            im_h = h_v[b, sl2]
            rv = tab_v[b, sl]
            re_t = t_v[b, sl]
            im_t = t_v[b, sl2]
            sn, cs = _sincos(rv * PHASE_SCALE)
            re_s = re_h * cs - im_h * sn
            im_s = re_h * sn + im_h * cs
            acc = acc + jnp.abs(re_s - re_t) + jnp.abs(im_s - im_t)
        plsc.store_compressed(o_v.at[pl.ds(b, L)],
                              MARGIN - plsc.cumsum(acc), mask=last)

    for c in copies[:3]:
        c.wait()

    @plsc.parallel_loop(0, HALF, 1, unroll=2)
    def _(b):
        one_elem(b)

    for c in copies[3:]:
        c.wait()

    @plsc.parallel_loop(HALF, BPW, 1, unroll=2)
    def _(b):
        one_elem(b)

    pltpu.sync_copy(o_v.at[pl.ds(0, BPW)], out_hbm.at[pl.ds(base, BPW)])


def kernel(sub, rel, obj, ent_emb, rel_emb):
    return _sc_score(sub.astype(jnp.int32), rel.astype(jnp.int32),
                     obj.astype(jnp.int32), ent_emb, rel_emb)


# speedup vs baseline: 1.0091x; 1.0091x over previous
import jax
import jax.numpy as jnp
from jax import lax
from jax.experimental import pallas as pl
from jax.experimental.pallas import tpu as pltpu
from jax.experimental.pallas import tpu_sc as plsc
import functools

NUM_ENT = 1000000
NUM_REL = 1000
D = 64
MARGIN = 12.0
BATCH = 4096
ERANGE = (MARGIN + 2.0) / D
PI = 3.141592653589793
PHASE_SCALE = PI / ERANGE

NC, NS, L = 2, 16, 16
NW = NC * NS
BPW = BATCH // NW

_SIN_C = (0.9994501730582466, -0.16583842947680993, 0.007998575320167352,
          -0.00014774043807849746)
_COS_C = (0.9999710932183866, -0.49983759608552286, 0.04152230455014086,
          -0.0013441068677407103, 1.906521608691092e-05)


def _sincos(ph):
    x2 = ph * ph
    s = _SIN_C[-1]
    for c in _SIN_C[-2::-1]:
        s = s * x2 + c
    s = s * ph
    c_ = _COS_C[-1]
    for c in _COS_C[-2::-1]:
        c_ = c_ * x2 + c
    return s, c_


_sc_mesh = plsc.VectorSubcoreMesh(core_axis_name="c", subcore_axis_name="s")


@functools.partial(
    pl.kernel,
    out_type=jax.ShapeDtypeStruct((BATCH,), jnp.float32),
    mesh=_sc_mesh,
    compiler_params=pltpu.CompilerParams(needs_layout_passes=False,
                                         use_tc_tiling_on_sc=False),
    scratch_types=[
        pltpu.VMEM((BPW,), jnp.int32),
        pltpu.VMEM((BPW,), jnp.int32),
        pltpu.VMEM((BPW,), jnp.int32),
        pltpu.VMEM((BPW, 2 * D), jnp.float32),
        pltpu.VMEM((BPW, 2 * D), jnp.float32),
        pltpu.VMEM((BPW, D), jnp.float32),
        pltpu.VMEM((BPW + L,), jnp.float32),
        pltpu.SemaphoreType.DMA,
        pltpu.SemaphoreType.DMA,
    ],
)
def _sc_score(sub_hbm, rel_hbm, obj_hbm, ent_hbm, rel_emb_hbm, out_hbm,
              sub_v, obj_v, rel_v, h_v, t_v, tab_v, o_v,
              sem_a, sem_b):
    wid = lax.axis_index("s") * NC + lax.axis_index("c")
    base = wid * BPW
    HALF = BPW // 2
    half = (pl.ds(0, HALF), pl.ds(HALF, HALF))
    copies = []
    for p, sem in ((0, sem_a), (1, sem_b)):
        hb = pl.ds(base + p * HALF, HALF)
        pltpu.sync_copy(sub_hbm.at[hb], sub_v.at[half[p]])
        pltpu.sync_copy(obj_hbm.at[hb], obj_v.at[half[p]])
        pltpu.sync_copy(rel_hbm.at[hb], rel_v.at[half[p]])
        copies += [
            pltpu.async_copy(ent_hbm.at[sub_v.at[half[p]]],
                             h_v.at[half[p]], sem),
            pltpu.async_copy(ent_hbm.at[obj_v.at[half[p]]],
                             t_v.at[half[p]], sem),
            pltpu.async_copy(rel_emb_hbm.at[rel_v.at[half[p]]],
                             tab_v.at[half[p]], sem),
        ]

    lane = lax.iota(jnp.int32, L)
    last = lane == (L - 1)

    def one_elem(b):
        acc = jnp.zeros((L,), jnp.float32)
        for k in range(D // L):
            sl = pl.ds(k * L, L)
            sl2 = pl.ds(D + k * L, L)
            re_h = h_v[b, sl]
            im_h = h_v[b, sl2]
            rv = tab_v[b, sl]
            re_t = t_v[b, sl]
            im_t = t_v[b, sl2]
            sn, cs = _sincos(rv * PHASE_SCALE)
            re_s = re_h * cs - im_h * sn
            im_s = re_h * sn + im_h * cs
            acc = acc + jnp.abs(re_s - re_t) + jnp.abs(im_s - im_t)
        plsc.store_compressed(o_v.at[pl.ds(b, L)],
                              MARGIN - plsc.cumsum(acc), mask=last)

    for c in copies[:3]:
        c.wait()

    @plsc.parallel_loop(0, HALF, 1, unroll=2)
    def _(b):
        one_elem(b)

    for c in copies[3:]:
        c.wait()

    @plsc.parallel_loop(HALF, BPW, 1, unroll=2)
    def _(b):
        one_elem(b)

    pltpu.sync_copy(o_v.at[pl.ds(0, BPW)], out_hbm.at[pl.ds(base, BPW)])


def kernel(sub, rel, obj, ent_emb, rel_emb):
    return _sc_score(sub.astype(jnp.int32), rel.astype(jnp.int32),
                     obj.astype(jnp.int32), ent_emb, rel_emb)
